# baseline (device time: 113926 ns/iter reference)
import jax
import jax.numpy as jnp
from jax import lax
from jax.experimental import pallas as pl
from jax.experimental.pallas import tpu as pltpu

M = 8192
N = 1024
HALF = M // 2
C = 16
R = HALF // C


def kernel(x):
    def body(
        x_hbm, out_hbm,
        ls, la, sx, rx, ob,
        ls_sem, la_sem, od_sem,
        xs_sem, xr_sem, fs_sem, fr_sem,
    ):
        my_x = lax.axis_index("x")
        my_y = lax.axis_index("y")
        ox = my_x * N
        oxo = (1 - my_x) * N
        rbd = my_y * HALF

        barrier = pltpu.get_barrier_semaphore()
        pl.semaphore_signal(
            barrier, inc=1, device_id=(1 - my_x, my_y),
            device_id_type=pl.DeviceIdType.MESH,
        )
        pl.semaphore_signal(
            barrier, inc=1, device_id=(my_x, 1 - my_y),
            device_id_type=pl.DeviceIdType.MESH,
        )
        pl.semaphore_wait(barrier, 2)

        def ls_copy(c, s):
            return pltpu.make_async_copy(
                x_hbm.at[0, pl.ds(rbd + c * R, R), pl.ds(oxo, N)],
                ls.at[s], ls_sem.at[s],
            )

        def la_copy(c, s):
            return pltpu.make_async_copy(
                x_hbm.at[0, pl.ds(rbd + c * R, R), pl.ds(ox, N)],
                la.at[s], la_sem.at[s],
            )

        def x_rdma(c):
            return pltpu.make_async_remote_copy(
                src_ref=sx.at[c], dst_ref=rx.at[c],
                send_sem=xs_sem.at[c], recv_sem=xr_sem.at[c],
                device_id=(1 - my_x, my_y),
                device_id_type=pl.DeviceIdType.MESH,
            )

        def f_rdma(c):
            return pltpu.make_async_remote_copy(
                src_ref=ob.at[c],
                dst_ref=out_hbm.at[pl.ds(rbd + c * R, R), :],
                send_sem=fs_sem.at[c], recv_sem=fr_sem.at[c],
                device_id=(my_x, 1 - my_y),
                device_id_type=pl.DeviceIdType.MESH,
            )

        def od_copy(c):
            return pltpu.make_async_copy(
                ob.at[c], out_hbm.at[pl.ds(rbd + c * R, R), :], od_sem.at[c]
            )

        ls_copy(0, 0).start()
        ls_copy(1, 1).start()
        for c in range(C):
            s = c % 2
            ls_copy(c, s).wait()
            sx[c] = ls[s].astype(jnp.bfloat16)
            x_rdma(c).start()
            if c + 2 < C:
                ls_copy(c + 2, s).start()

        la_copy(0, 0).start()
        la_copy(1, 1).start()
        for c in range(C):
            s = c % 2
            x_rdma(c).wait_recv()
            la_copy(c, s).wait()
            ob[c] = (la[s] + rx[c].astype(jnp.float32)).astype(jnp.bfloat16)
            f_rdma(c).start()
            od_copy(c).start()
            if c + 2 < C:
                la_copy(c + 2, s).start()

        for c in range(C):
            od_copy(c).wait()
            x_rdma(c).wait_send()
            f_rdma(c).wait_send()
            f_rdma(c).wait_recv()

    return pl.pallas_call(
        body,
        in_specs=[pl.BlockSpec(memory_space=pl.MemorySpace.ANY)],
        out_specs=pl.BlockSpec(memory_space=pl.MemorySpace.ANY),
        out_shape=jax.ShapeDtypeStruct((M, N), jnp.bfloat16),
        scratch_shapes=[
            pltpu.VMEM((2, R, N), jnp.float32),
            pltpu.VMEM((2, R, N), jnp.float32),
            pltpu.VMEM((C, R, N), jnp.bfloat16),
            pltpu.VMEM((C, R, N), jnp.bfloat16),
            pltpu.VMEM((C, R, N), jnp.bfloat16),
            pltpu.SemaphoreType.DMA((2,)),
            pltpu.SemaphoreType.DMA((2,)),
            pltpu.SemaphoreType.DMA((C,)),
            pltpu.SemaphoreType.DMA((C,)),
            pltpu.SemaphoreType.DMA((C,)),
            pltpu.SemaphoreType.DMA((C,)),
            pltpu.SemaphoreType.DMA((C,)),
        ],
        compiler_params=pltpu.CompilerParams(collective_id=0),
    )(x)


# device time: 113878 ns/iter; 1.0004x vs baseline; 1.0004x over previous
import jax
import jax.numpy as jnp
from jax import lax
from jax.experimental import pallas as pl
from jax.experimental.pallas import tpu as pltpu

M = 8192
N = 1024
HALF = M // 2
C = 16
R = HALF // C


def kernel(x):
    def body(
        x_hbm, out_hbm,
        ls, la, sx, rx, ob,
        ls_sem, la_sem, od_sem,
        xs_sem, xr_sem, fs_sem, fr_sem,
    ):
        my_x = lax.axis_index("x")
        my_y = lax.axis_index("y")
        ox = my_x * N
        oxo = (1 - my_x) * N
        rbd = my_y * HALF

        barrier = pltpu.get_barrier_semaphore()
        pl.semaphore_signal(
            barrier, inc=1, device_id=(1 - my_x, my_y),
            device_id_type=pl.DeviceIdType.MESH,
        )
        pl.semaphore_signal(
            barrier, inc=1, device_id=(my_x, 1 - my_y),
            device_id_type=pl.DeviceIdType.MESH,
        )
        pl.semaphore_wait(barrier, 2)

        def ls_copy(c, s):
            return pltpu.make_async_copy(
                x_hbm.at[0, pl.ds(rbd + c * R, R), pl.ds(oxo, N)],
                ls.at[s], ls_sem.at[s],
            )

        def la_copy(c, s):
            return pltpu.make_async_copy(
                x_hbm.at[0, pl.ds(rbd + c * R, R), pl.ds(ox, N)],
                la.at[s], la_sem.at[s],
            )

        def x_rdma(c):
            return pltpu.make_async_remote_copy(
                src_ref=sx.at[c], dst_ref=rx.at[c],
                send_sem=xs_sem.at[c], recv_sem=xr_sem.at[c],
                device_id=(1 - my_x, my_y),
                device_id_type=pl.DeviceIdType.MESH,
            )

        def f_rdma(c):
            return pltpu.make_async_remote_copy(
                src_ref=ob.at[c],
                dst_ref=out_hbm.at[pl.ds(rbd + c * R, R), :],
                send_sem=fs_sem.at[c], recv_sem=fr_sem.at[c],
                device_id=(my_x, 1 - my_y),
                device_id_type=pl.DeviceIdType.MESH,
            )

        def od_copy(c):
            return pltpu.make_async_copy(
                ob.at[c], out_hbm.at[pl.ds(rbd + c * R, R), :], od_sem.at[c]
            )

        ls_copy(0, 0).start()
        ls_copy(1, 1).start()
        for c in range(C):
            s = c % 2
            ls_copy(c, s).wait()
            sx[c] = ls[s].astype(jnp.bfloat16)
            x_rdma(c).start()
            if c + 2 < C:
                ls_copy(c + 2, s).start()

        la_copy(0, 0).start()
        la_copy(1, 1).start()
        for c in range(C):
            s = c % 2
            x_rdma(c).wait_recv()
            la_copy(c, s).wait()
            ob[c] = (la[s] + rx[c].astype(jnp.float32)).astype(jnp.bfloat16)
            f_rdma(c).start()
            od_copy(c).start()
            if c + 2 < C:
                la_copy(c + 2, s).start()

        for c in range(C):
            od_copy(c).wait()
            x_rdma(c).wait_send()
            f_rdma(c).wait_send()
            f_rdma(c).wait_recv()

    return pl.pallas_call(
        body,
        in_specs=[pl.BlockSpec(memory_space=pl.MemorySpace.ANY)],
        out_specs=pl.BlockSpec(memory_space=pltpu.MemorySpace.HBM),
        out_shape=jax.ShapeDtypeStruct((M, N), jnp.bfloat16),
        scratch_shapes=[
            pltpu.VMEM((2, R, N), jnp.float32),
            pltpu.VMEM((2, R, N), jnp.float32),
            pltpu.VMEM((C, R, N), jnp.bfloat16),
            pltpu.VMEM((C, R, N), jnp.bfloat16),
            pltpu.VMEM((C, R, N), jnp.bfloat16),
            pltpu.SemaphoreType.DMA((2,)),
            pltpu.SemaphoreType.DMA((2,)),
            pltpu.SemaphoreType.DMA((C,)),
            pltpu.SemaphoreType.DMA((C,)),
            pltpu.SemaphoreType.DMA((C,)),
            pltpu.SemaphoreType.DMA((C,)),
            pltpu.SemaphoreType.DMA((C,)),
        ],
        compiler_params=pltpu.CompilerParams(collective_id=0),
    )(x)
